# baseline (speedup 1.0000x reference)
import functools
import math

import jax
import jax.numpy as jnp
from jax import lax
from jax.experimental import pallas as pl
from jax.experimental.pallas import tpu as pltpu
from jax.experimental.pallas import tpu_sc as plsc

V = 10000
E = 160000
F = 64
T = 4
NUM_TYPES = 100
CUTOFF = 30.0
GAP = 0.5
NCEN = 60
LOG2 = math.log(2.0)

VP = 10240
EPAD = 163840
CHUNK = 64
N2 = VP * F

EB = 2048
BB = 512
NB = 4096


def _ssp(x):
    return jnp.maximum(x, 0.0) + jnp.log(1.0 + jnp.exp(-jnp.abs(x))) - LOG2


def _prep_body(emb_ref, wn_ref, bn_ref, out_ref):
    out_ref[:, pl.ds(0, F)] = (
        jnp.dot(emb_ref[...], wn_ref[...], preferred_element_type=jnp.float32)
        + bn_ref[...])
    out_ref[:, pl.ds(F, F)] = jnp.zeros((NUM_TYPES, F), jnp.float32)


def _rbf(d_ref):
    d = jnp.broadcast_to(d_ref[...], (EB, F))
    cen = (lax.broadcasted_iota(jnp.int32, (EB, F), 1).astype(jnp.float32)
           * (CUTOFF / (NCEN - 1)))
    return jnp.exp((-1.0 / GAP) * (d - cen) * (d - cen))


def _edge1_body(d_ref, we1_ref, be1_ref, we2_ref, be2_ref, he_ref):
    ex = _rbf(d_ref)
    h1 = _ssp(jnp.dot(ex, we1_ref[...], preferred_element_type=jnp.float32)
              + be1_ref[...])
    he_ref[...] = _ssp(
        jnp.dot(h1, we2_ref[...], preferred_element_type=jnp.float32)
        + be2_ref[...])


def _edge2_body(d_ref, we1_ref, be1_ref, we2_ref, be2_ref, he1_ref, he2_ref):
    ex = _rbf(d_ref)
    for l, he_ref in ((0, he1_ref), (1, he2_ref)):
        h1 = _ssp(jnp.dot(ex, we1_ref[l], preferred_element_type=jnp.float32)
                  + be1_ref[l])
        he_ref[...] = _ssp(
            jnp.dot(h1, we2_ref[l], preferred_element_type=jnp.float32)
            + be2_ref[l])


def _b0_body(h_ref, wc_ref, bc_ref, wo_ref, bo_ref, o_ref):
    h = h_ref[0, :, pl.ds(0, F)] + h_ref[1, :, pl.ds(0, F)]
    a = _ssp(jnp.dot(h, wc_ref[...], preferred_element_type=jnp.float32)
             + bc_ref[...])
    o = jnp.dot(a, wo_ref[...], preferred_element_type=jnp.float32) + bo_ref[...]
    for t in range(T):
        o_ref[t] = o


def _b_body(h_ref, wc_ref, bc_ref, wo_ref, bo_ref, o_ref):
    for t in range(T):
        h = h_ref[t // 2, :, pl.ds(64 * (t % 2), 64)]
        a = _ssp(jnp.dot(h, wc_ref[...], preferred_element_type=jnp.float32)
                 + bc_ref[...])
        o_ref[t] = (jnp.dot(a, wo_ref[...], preferred_element_type=jnp.float32)
                    + bo_ref[...])


def _tm_body(x_ref, w1t_ref, b1_ref, w2t_ref, b2_ref, y_ref):
    z = _ssp(jnp.dot(w1t_ref[...], x_ref[...],
                     preferred_element_type=jnp.float32) + b1_ref[...])
    y_ref[...] = (jnp.dot(w2t_ref[...], z, preferred_element_type=jnp.float32)
                  + b2_ref[...])


def _tmf_body(x_ref, w1t_ref, b1_ref, w2t_ref, b2_ref,
              wr1t_ref, br1_ref, wr2t_ref, br2_ref, out_ref):
    z = _ssp(jnp.dot(w1t_ref[...], x_ref[...],
                     preferred_element_type=jnp.float32) + b1_ref[...])
    y = (jnp.dot(w2t_ref[...], z, preferred_element_type=jnp.float32)
         + b2_ref[...])
    r = _ssp(jnp.dot(wr1t_ref[...], y, preferred_element_type=jnp.float32)
             + br1_ref[...])
    out_ref[...] = (jnp.dot(wr2t_ref[...], r,
                            preferred_element_type=jnp.float32) + br2_ref[...])


def _proj_body(y_ref, wn_ref, bn_ref, hv_ref):
    for t in range(T):
        hv = (jnp.dot(y_ref[t], wn_ref[...], preferred_element_type=jnp.float32)
              + bn_ref[...])
        hv_ref[t // 2, :, pl.ds(64 * (t % 2), 64)] = hv


def _full(shape):
    nd = len(shape)
    return pl.BlockSpec(shape, lambda i: (0,) * nd)


@functools.lru_cache(maxsize=None)
def _sc_kernels():
    mesh = plsc.VectorSubcoreMesh(core_axis_name="c", subcore_axis_name="s")
    rpt = VP // 16

    @functools.partial(
        pl.kernel, mesh=mesh,
        out_type=jax.ShapeDtypeStruct((2 * VP, 2 * F), jnp.float32),
        scratch_types=[
            pltpu.VMEM((64,), jnp.int32),
            pltpu.VMEM((64, 2 * F), jnp.float32),
            pltpu.SemaphoreType.DMA,
        ])
    def embed_k(table_hbm, nt_hbm, out_hbm, ntb, rb, sem):
        wid = lax.axis_index("s") * 2 + lax.axis_index("c")
        base = wid * (VP // 32)

        def f(j, c):
            b0 = base + j * 64
            pltpu.sync_copy(nt_hbm.at[pl.ds(b0, 64)], ntb)
            pltpu.async_copy(table_hbm.at[ntb], rb, sem).wait()
            pltpu.sync_copy(rb, out_hbm.at[pl.ds(b0, 64)])
            pltpu.sync_copy(rb, out_hbm.at[pl.ds(VP + b0, 64)])
            return c

        lax.fori_loop(0, (VP // 32) // 64, f, 0)

    def make_msg(ncht, he_per_core):
        width = 2 * F
        nch = ncht // 16
        SB = 16
        nsb = nch // SB

        @functools.partial(
            pl.kernel, mesh=mesh,
            out_type=jax.ShapeDtypeStruct((2, VP, width), jnp.float32),
            scratch_types=[
                pltpu.VMEM((2, SB, CHUNK), jnp.int32),
                pltpu.VMEM((2, SB, CHUNK), jnp.int32),
                pltpu.VMEM((CHUNK, width), jnp.float32),
                pltpu.VMEM((CHUNK, width), jnp.float32),
                pltpu.VMEM((CHUNK, F), jnp.float32),
                pltpu.VMEM((CHUNK, F), jnp.float32),
                pltpu.VMEM_SHARED((VP, width), jnp.float32),
                pltpu.SemaphoreType.DMA,
                pltpu.SemaphoreType.DMA,
                pltpu.SemaphoreType.DMA,
                pltpu.SemaphoreType.DMA,
                pltpu.SemaphoreType.DMA,
                pltpu.SemaphoreType.DMA,
                pltpu.SemaphoreType.DMA,
                pltpu.SemaphoreType.DMA,
                pltpu.SemaphoreType.DMA,
                pltpu.SemaphoreType.DMA,
            ])
        def msg_k(hv_hbm, he_hbm, src_hbm, dst_hbm, out_hbm,
                  sidx, didx, rows0, rows1, heb0, heb1, acc,
                  sg0, sg1, sh0, sh1, ss0, ss1, sis0, sis1, sid0, sid1):
            cid = lax.axis_index("c")
            sid = lax.axis_index("s")
            rows = (rows0, rows1)
            heb = (heb0, heb1)
            sg = (sg0, sg1)
            sh = (sh0, sh1)
            ss = (ss0, ss1)
            sis = (sis0, sis1)
            sdd = (sid0, sid1)
            cbase = sid * nch
            ebase = cbase * CHUNK

            def fetch_idx(b, ip):
                pltpu.async_copy(src_hbm.at[cid, pl.ds(cbase + b * SB, SB)],
                                 sidx.at[ip], sis[ip])
                pltpu.async_copy(dst_hbm.at[cid, pl.ds(cbase + b * SB, SB)],
                                 didx.at[ip], sdd[ip])

            def wait_idx(ip):
                pltpu.make_async_copy(src_hbm.at[cid, pl.ds(0, SB)],
                                      sidx.at[ip], sis[ip]).wait()
                pltpu.make_async_copy(dst_hbm.at[cid, pl.ds(0, SB)],
                                      didx.at[ip], sdd[ip]).wait()

            def he_at(off, n):
                if he_per_core:
                    return he_hbm.at[cid, pl.ds(off, n)]
                return he_hbm.at[pl.ds(off, n)]

            def start(g, par):
                ip = (g // SB) % 2
                pltpu.async_copy(hv_hbm.at[sidx.at[ip, g % SB]], rows[par],
                                 sg[par])
                pltpu.async_copy(he_at(ebase + g * CHUNK, CHUNK),
                                 heb[par], sh[par])

            def wait_in(par):
                pltpu.make_async_copy(hv_hbm.at[sidx.at[0, 0]], rows[par],
                                      sg[par]).wait()
                pltpu.make_async_copy(he_at(0, CHUNK), heb[par],
                                      sh[par]).wait()

            def drain_sc(par):
                pltpu.make_async_copy(rows[par], acc.at[didx.at[0, 0]],
                                      ss[par]).wait()

            for r in range(CHUNK):
                for q in range(width // 16):
                    rows0[r, pl.ds(16 * q, 16)] = jnp.zeros((16,), jnp.float32)

            def zf(j, c):
                pltpu.sync_copy(rows0,
                                acc.at[pl.ds(sid * rpt + j * CHUNK, CHUNK)])
                return c

            lax.fori_loop(0, rpt // CHUNK, zf, 0)
            plsc.subcore_barrier()

            fetch_idx(0, 0)
            wait_idx(0)
            start(0, 0)

            def step(g, c):
                par = lax.rem(g, 2)

                def m_par(par):
                    wait_in(par)

                    def rowf(r4, c2):
                        for u in range(4):
                            r = r4 * 4 + u
                            hr = [heb[par][r, pl.ds(16 * k, 16)]
                                  for k in range(4)]
                            for k in range(width // 16):
                                rows[par][r, pl.ds(16 * k, 16)] = (
                                    rows[par][r, pl.ds(16 * k, 16)] * hr[k % 4])
                        return c2

                    lax.fori_loop(0, CHUNK // 4, rowf, 0)
                    ip = (g // SB) % 2
                    pltpu.async_copy(rows[par], acc.at[didx.at[ip, g % SB]],
                                     ss[par], add=True)

                    @pl.when(g + 1 < nch)
                    def _():
                        @pl.when(g >= 1)
                        def _():
                            drain_sc(1 - par)

                        nb_par = lax.rem((g + 1) // SB, 2)
                        for ipc in (0, 1):
                            @pl.when(jnp.logical_and(
                                lax.rem(g + 1, SB) == 0, nb_par == ipc))
                            def _(ipc=ipc):
                                wait_idx(ipc)

                        start(g + 1, 1 - par)

                    fb = g // SB + 1
                    fetch_now = jnp.logical_and(lax.rem(g, SB) == 2, fb < nsb)
                    for ipc in (0, 1):
                        @pl.when(jnp.logical_and(fetch_now,
                                                 lax.rem(fb, 2) == ipc))
                        def _(ipc=ipc):
                            fetch_idx(fb, ipc)

                @pl.when(par == 0)
                def _():
                    m_par(0)

                @pl.when(par == 1)
                def _():
                    m_par(1)

                return c

            lax.fori_loop(0, nch, step, 0)
            drain_sc(0)
            drain_sc(1)
            plsc.subcore_barrier()
            pltpu.sync_copy(acc.at[pl.ds(sid * rpt, rpt)],
                            out_hbm.at[cid, pl.ds(sid * rpt, rpt)])

        return msg_k

    return (embed_k, make_msg(EPAD // CHUNK, False),
            make_msg(EPAD // CHUNK // 2, True))


def kernel(node_types, edge_dists, edge_index, params):
    f32 = jnp.float32
    i32 = jnp.int32
    src = edge_index[0]
    dst = edge_index[1]
    srcp = jnp.concatenate([src, jnp.zeros((EPAD - E,), i32)])
    dstp = jnp.concatenate([dst, jnp.full((EPAD - E,), V, i32)])
    src_c = srcp.reshape(EPAD // CHUNK, CHUNK)
    dst_c = dstp.reshape(EPAD // CHUNK, CHUNK)
    srcs1 = jnp.stack([src_c, src_c + VP])
    dsts1 = jnp.stack([dst_c, dst_c])
    nh = EPAD // CHUNK // 2
    srcs0 = jnp.stack([src_c[:nh], src_c[nh:] + VP])
    dsts0 = jnp.stack([dst_c[:nh], dst_c[nh:]])
    distp = jnp.concatenate([edge_dists.astype(f32),
                             jnp.zeros((EPAD - E, 1), f32)], axis=0)
    ntp = jnp.concatenate([node_types, jnp.zeros((VP - V,), i32)])

    layers = params['layers']
    we1 = jnp.stack([jnp.pad(p['We1'], ((0, F - NCEN), (0, 0)))
                     for p in layers[1:]])
    be1 = jnp.stack([p['be1'].reshape(1, F) for p in layers[1:]])
    we2 = jnp.stack([p['We2'] for p in layers[1:]])
    be2 = jnp.stack([p['be2'].reshape(1, F) for p in layers[1:]])

    embed_k, msg_k, msg0_k = _sc_kernels()

    p0 = layers[0]
    tableP = pl.pallas_call(
        _prep_body,
        grid=(1,),
        in_specs=[_full((NUM_TYPES, F)), _full((F, F)), _full((1, F))],
        out_specs=_full((NUM_TYPES, 2 * F)),
        out_shape=jax.ShapeDtypeStruct((NUM_TYPES, 2 * F), f32),
    )(params['embed'], p0['Wn'], p0['bn'].reshape(1, F))

    hv = embed_k(tableP, ntp)

    he0 = pl.pallas_call(
        _edge1_body,
        grid=(EPAD // EB,),
        in_specs=[
            pl.BlockSpec((EB, 1), lambda i: (i, 0)),
            _full((F, F)), _full((1, F)), _full((F, F)), _full((1, F)),
        ],
        out_specs=pl.BlockSpec((EB, F), lambda i: (i, 0)),
        out_shape=jax.ShapeDtypeStruct((EPAD, F), f32),
    )(distp, jnp.pad(p0['We1'], ((0, F - NCEN), (0, 0))),
      p0['be1'].reshape(1, F), p0['We2'], p0['be2'].reshape(1, F))

    he1, he2 = pl.pallas_call(
        _edge2_body,
        grid=(EPAD // EB,),
        in_specs=[
            pl.BlockSpec((EB, 1), lambda i: (i, 0)),
            _full((2, F, F)), _full((2, 1, F)), _full((2, F, F)),
            _full((2, 1, F)),
        ],
        out_specs=[pl.BlockSpec((EB, F), lambda i: (i, 0))] * 2,
        out_shape=[jax.ShapeDtypeStruct((EPAD, F), f32)] * 2,
    )(distp, we1, be1, we2, be2)
    he = {0: he0, 1: he1, 2: he2}

    def tc_b0(h, p):
        return pl.pallas_call(
            _b0_body,
            grid=(VP // BB,),
            in_specs=[
                pl.BlockSpec((1, BB, 2 * F), lambda i: (0, i, 0)),
                _full((F, F)), _full((1, F)), _full((F, F)), _full((1, F)),
            ],
            out_specs=pl.BlockSpec((T, BB, F), lambda i: (0, i, 0)),
            out_shape=jax.ShapeDtypeStruct((T, VP, F), f32),
        )(h, p['Wc'], p['bc'].reshape(1, F), p['Wo'], p['bo'].reshape(1, F))

    def tc_b(h, p):
        return pl.pallas_call(
            _b_body,
            grid=(VP // BB,),
            in_specs=[
                pl.BlockSpec((2, BB, 2 * F), lambda i: (0, i, 0)),
                _full((F, F)), _full((1, F)), _full((F, F)), _full((1, F)),
            ],
            out_specs=pl.BlockSpec((T, BB, F), lambda i: (0, i, 0)),
            out_shape=jax.ShapeDtypeStruct((T, VP, F), f32),
        )(h, p['Wc'], p['bc'].reshape(1, F), p['Wo'], p['bo'].reshape(1, F))

    def tc_tm(o4, p):
        x2 = o4.reshape(T, N2)
        y = pl.pallas_call(
            _tm_body,
            grid=(N2 // NB,),
            in_specs=[
                pl.BlockSpec((T, NB), lambda i: (0, i)),
                _full((F, T)), _full((F, 1)), _full((T, F)), _full((T, 1)),
            ],
            out_specs=pl.BlockSpec((T, NB), lambda i: (0, i)),
            out_shape=jax.ShapeDtypeStruct((T, N2), f32),
        )(x2, p['Wt1'].T, p['bt1'].reshape(F, 1), p['Wt2'].T,
          p['bt2'].reshape(T, 1))
        return y.reshape(T, VP, F)

    def tc_proj(y4, p):
        return pl.pallas_call(
            _proj_body,
            grid=(VP // BB,),
            in_specs=[
                pl.BlockSpec((T, BB, F), lambda i: (0, i, 0)),
                _full((F, F)), _full((1, F)),
            ],
            out_specs=pl.BlockSpec((2, BB, 2 * F), lambda i: (0, i, 0)),
            out_shape=jax.ShapeDtypeStruct((2, VP, 2 * F), f32),
        )(y4, p['Wn'], p['bn'].reshape(1, F))

    he0c = he0.reshape(2, EPAD // 2, F)
    h0 = msg0_k(hv, he0c, srcs0, dsts0)
    y = tc_tm(tc_b0(h0, p0), p0)

    for l in (1, 2):
        p = layers[l]
        hv2 = tc_proj(y, p).reshape(2 * VP, 2 * F)
        hl = msg_k(hv2, he[l], srcs1, dsts1)
        if l == 1:
            y = tc_tm(tc_b(hl, p), p)
        else:
            o4 = tc_b(hl, p)

    x2 = o4.reshape(T, N2)
    p2 = layers[2]
    out = pl.pallas_call(
        _tmf_body,
        grid=(N2 // NB,),
        in_specs=[
            pl.BlockSpec((T, NB), lambda i: (0, i)),
            _full((F, T)), _full((F, 1)), _full((T, F)), _full((T, 1)),
            _full((32, T)), _full((32, 1)), _full((1, 32)), _full((1, 1)),
        ],
        out_specs=pl.BlockSpec((1, NB), lambda i: (0, i)),
        out_shape=jax.ShapeDtypeStruct((1, N2), f32),
    )(x2, p2['Wt1'].T, p2['bt1'].reshape(F, 1), p2['Wt2'].T,
      p2['bt2'].reshape(T, 1), params['Wr1'].T, params['br1'].reshape(32, 1),
      params['Wr2'].T, params['br2'].reshape(1, 1))

    return out.reshape(VP, F)[:V]


# speedup vs baseline: 1.0329x; 1.0329x over previous
import functools
import math

import jax
import jax.numpy as jnp
from jax import lax
from jax.experimental import pallas as pl
from jax.experimental.pallas import tpu as pltpu
from jax.experimental.pallas import tpu_sc as plsc

V = 10000
E = 160000
F = 64
T = 4
NUM_TYPES = 100
CUTOFF = 30.0
GAP = 0.5
NCEN = 60
LOG2 = math.log(2.0)

VP = 10240
EPAD = 163840
CHUNK = 64
N2 = VP * F

EB = 1600
BB = 512
NB = 4096


def _ssp(x):
    return jnp.maximum(x, 0.0) + jnp.log(1.0 + jnp.exp(-jnp.abs(x))) - LOG2


def _prep_body(emb_ref, wn_ref, bn_ref, out_ref):
    out_ref[:, pl.ds(0, F)] = (
        jnp.dot(emb_ref[...], wn_ref[...], preferred_element_type=jnp.float32)
        + bn_ref[...])
    out_ref[:, pl.ds(F, F)] = jnp.zeros((NUM_TYPES, F), jnp.float32)


def _rbf(d_ref):
    d = jnp.broadcast_to(d_ref[...], (EB, F))
    cen = (lax.broadcasted_iota(jnp.int32, (EB, F), 1).astype(jnp.float32)
           * (CUTOFF / (NCEN - 1)))
    return jnp.exp((-1.0 / GAP) * (d - cen) * (d - cen))


def _edge1_body(d_ref, we1_ref, be1_ref, we2_ref, be2_ref, he_ref):
    ex = _rbf(d_ref)
    h1 = _ssp(jnp.dot(ex, we1_ref[...], preferred_element_type=jnp.float32)
              + be1_ref[...])
    he_ref[...] = _ssp(
        jnp.dot(h1, we2_ref[...], preferred_element_type=jnp.float32)
        + be2_ref[...])


def _edge2_body(d_ref, we1_ref, be1_ref, we2_ref, be2_ref, he1_ref, he2_ref):
    ex = _rbf(d_ref)
    for l, he_ref in ((0, he1_ref), (1, he2_ref)):
        h1 = _ssp(jnp.dot(ex, we1_ref[l], preferred_element_type=jnp.float32)
                  + be1_ref[l])
        he_ref[...] = _ssp(
            jnp.dot(h1, we2_ref[l], preferred_element_type=jnp.float32)
            + be2_ref[l])


def _b0_body(h_ref, wc_ref, bc_ref, wo_ref, bo_ref, o_ref):
    h = h_ref[0, :, pl.ds(0, F)] + h_ref[1, :, pl.ds(0, F)]
    a = _ssp(jnp.dot(h, wc_ref[...], preferred_element_type=jnp.float32)
             + bc_ref[...])
    o = jnp.dot(a, wo_ref[...], preferred_element_type=jnp.float32) + bo_ref[...]
    for t in range(T):
        o_ref[t] = o


def _b_body(h_ref, wc_ref, bc_ref, wo_ref, bo_ref, o_ref):
    for t in range(T):
        h = h_ref[t // 2, :, pl.ds(64 * (t % 2), 64)]
        a = _ssp(jnp.dot(h, wc_ref[...], preferred_element_type=jnp.float32)
                 + bc_ref[...])
        o_ref[t] = (jnp.dot(a, wo_ref[...], preferred_element_type=jnp.float32)
                    + bo_ref[...])


def _tm_body(x_ref, w1t_ref, b1_ref, w2t_ref, b2_ref, y_ref):
    z = _ssp(jnp.dot(w1t_ref[...], x_ref[...],
                     preferred_element_type=jnp.float32) + b1_ref[...])
    y_ref[...] = (jnp.dot(w2t_ref[...], z, preferred_element_type=jnp.float32)
                  + b2_ref[...])


def _tmf_body(x_ref, w1t_ref, b1_ref, w2t_ref, b2_ref,
              wr1t_ref, br1_ref, wr2t_ref, br2_ref, out_ref):
    z = _ssp(jnp.dot(w1t_ref[...], x_ref[...],
                     preferred_element_type=jnp.float32) + b1_ref[...])
    y = (jnp.dot(w2t_ref[...], z, preferred_element_type=jnp.float32)
         + b2_ref[...])
    r = _ssp(jnp.dot(wr1t_ref[...], y, preferred_element_type=jnp.float32)
             + br1_ref[...])
    out_ref[...] = (jnp.dot(wr2t_ref[...], r,
                            preferred_element_type=jnp.float32) + br2_ref[...])


def _proj_body(y_ref, wn_ref, bn_ref, hv_ref):
    for t in range(T):
        hv = (jnp.dot(y_ref[t], wn_ref[...], preferred_element_type=jnp.float32)
              + bn_ref[...])
        hv_ref[t // 2, :, pl.ds(64 * (t % 2), 64)] = hv


def _full(shape):
    nd = len(shape)
    return pl.BlockSpec(shape, lambda i: (0,) * nd)


@functools.lru_cache(maxsize=None)
def _sc_kernels():
    mesh = plsc.VectorSubcoreMesh(core_axis_name="c", subcore_axis_name="s")
    rpt = VP // 16

    @functools.partial(
        pl.kernel, mesh=mesh,
        out_type=jax.ShapeDtypeStruct((2 * VP, 2 * F), jnp.float32),
        scratch_types=[
            pltpu.VMEM((64,), jnp.int32),
            pltpu.VMEM((64, 2 * F), jnp.float32),
            pltpu.SemaphoreType.DMA,
        ])
    def embed_k(table_hbm, nt_hbm, out_hbm, ntb, rb, sem):
        wid = lax.axis_index("s") * 2 + lax.axis_index("c")
        base = wid * (VP // 32)

        def f(j, c):
            b0 = base + j * 64
            pltpu.sync_copy(nt_hbm.at[pl.ds(b0, 64)], ntb)
            pltpu.async_copy(table_hbm.at[ntb], rb, sem).wait()
            pltpu.sync_copy(rb, out_hbm.at[pl.ds(b0, 64)])
            pltpu.sync_copy(rb, out_hbm.at[pl.ds(VP + b0, 64)])
            return c

        lax.fori_loop(0, (VP // 32) // 64, f, 0)

    def make_msg(ncht, he_per_core):
        width = 2 * F
        nch = ncht // 16
        SB = 16
        nsb = nch // SB

        @functools.partial(
            pl.kernel, mesh=mesh,
            out_type=jax.ShapeDtypeStruct((2, VP, width), jnp.float32),
            scratch_types=[
                pltpu.VMEM((2, SB, CHUNK), jnp.int32),
                pltpu.VMEM((2, SB, CHUNK), jnp.int32),
                pltpu.VMEM((CHUNK, width), jnp.float32),
                pltpu.VMEM((CHUNK, width), jnp.float32),
                pltpu.VMEM((CHUNK, F), jnp.float32),
                pltpu.VMEM((CHUNK, F), jnp.float32),
                pltpu.VMEM_SHARED((VP, width), jnp.float32),
                pltpu.SemaphoreType.DMA,
                pltpu.SemaphoreType.DMA,
                pltpu.SemaphoreType.DMA,
                pltpu.SemaphoreType.DMA,
                pltpu.SemaphoreType.DMA,
                pltpu.SemaphoreType.DMA,
                pltpu.SemaphoreType.DMA,
                pltpu.SemaphoreType.DMA,
                pltpu.SemaphoreType.DMA,
                pltpu.SemaphoreType.DMA,
            ])
        def msg_k(hv_hbm, he_hbm, src_hbm, dst_hbm, out_hbm,
                  sidx, didx, rows0, rows1, heb0, heb1, acc,
                  sg0, sg1, sh0, sh1, ss0, ss1, sis0, sis1, sid0, sid1):
            cid = lax.axis_index("c")
            sid = lax.axis_index("s")
            rows = (rows0, rows1)
            heb = (heb0, heb1)
            sg = (sg0, sg1)
            sh = (sh0, sh1)
            ss = (ss0, ss1)
            sis = (sis0, sis1)
            sdd = (sid0, sid1)
            cbase = sid * nch
            ebase = cbase * CHUNK

            def fetch_idx(b, ip):
                pltpu.async_copy(src_hbm.at[cid, pl.ds(cbase + b * SB, SB)],
                                 sidx.at[ip], sis[ip])
                pltpu.async_copy(dst_hbm.at[cid, pl.ds(cbase + b * SB, SB)],
                                 didx.at[ip], sdd[ip])

            def wait_idx(ip):
                pltpu.make_async_copy(src_hbm.at[cid, pl.ds(0, SB)],
                                      sidx.at[ip], sis[ip]).wait()
                pltpu.make_async_copy(dst_hbm.at[cid, pl.ds(0, SB)],
                                      didx.at[ip], sdd[ip]).wait()

            def he_at(off, n):
                if he_per_core:
                    return he_hbm.at[cid, pl.ds(off, n)]
                return he_hbm.at[pl.ds(off, n)]

            def start(g, par):
                ip = (g // SB) % 2
                pltpu.async_copy(hv_hbm.at[sidx.at[ip, g % SB]], rows[par],
                                 sg[par])
                pltpu.async_copy(he_at(ebase + g * CHUNK, CHUNK),
                                 heb[par], sh[par])

            def wait_in(par):
                pltpu.make_async_copy(hv_hbm.at[sidx.at[0, 0]], rows[par],
                                      sg[par]).wait()
                pltpu.make_async_copy(he_at(0, CHUNK), heb[par],
                                      sh[par]).wait()

            def drain_sc(par):
                pltpu.make_async_copy(rows[par], acc.at[didx.at[0, 0]],
                                      ss[par]).wait()

            for r in range(CHUNK):
                for q in range(width // 16):
                    rows0[r, pl.ds(16 * q, 16)] = jnp.zeros((16,), jnp.float32)

            def zf(j, c):
                pltpu.sync_copy(rows0,
                                acc.at[pl.ds(sid * rpt + j * CHUNK, CHUNK)])
                return c

            lax.fori_loop(0, rpt // CHUNK, zf, 0)
            plsc.subcore_barrier()

            fetch_idx(0, 0)
            wait_idx(0)
            start(0, 0)

            def step(g, c):
                par = lax.rem(g, 2)

                def m_par(par):
                    wait_in(par)

                    def rowf(r4, c2):
                        for u in range(4):
                            r = r4 * 4 + u
                            hr = [heb[par][r, pl.ds(16 * k, 16)]
                                  for k in range(4)]
                            for k in range(width // 16):
                                rows[par][r, pl.ds(16 * k, 16)] = (
                                    rows[par][r, pl.ds(16 * k, 16)] * hr[k % 4])
                        return c2

                    lax.fori_loop(0, CHUNK // 4, rowf, 0)
                    ip = (g // SB) % 2
                    pltpu.async_copy(rows[par], acc.at[didx.at[ip, g % SB]],
                                     ss[par], add=True)

                    @pl.when(g + 1 < nch)
                    def _():
                        @pl.when(g >= 1)
                        def _():
                            drain_sc(1 - par)

                        nb_par = lax.rem((g + 1) // SB, 2)
                        for ipc in (0, 1):
                            @pl.when(jnp.logical_and(
                                lax.rem(g + 1, SB) == 0, nb_par == ipc))
                            def _(ipc=ipc):
                                wait_idx(ipc)

                        start(g + 1, 1 - par)

                    fb = g // SB + 1
                    fetch_now = jnp.logical_and(lax.rem(g, SB) == 2, fb < nsb)
                    for ipc in (0, 1):
                        @pl.when(jnp.logical_and(fetch_now,
                                                 lax.rem(fb, 2) == ipc))
                        def _(ipc=ipc):
                            fetch_idx(fb, ipc)

                @pl.when(par == 0)
                def _():
                    m_par(0)

                @pl.when(par == 1)
                def _():
                    m_par(1)

                return c

            lax.fori_loop(0, nch, step, 0)
            drain_sc(0)
            drain_sc(1)
            plsc.subcore_barrier()
            pltpu.sync_copy(acc.at[pl.ds(sid * rpt, rpt)],
                            out_hbm.at[cid, pl.ds(sid * rpt, rpt)])

        return msg_k

    return (embed_k, make_msg(EPAD // CHUNK, False),
            make_msg(EPAD // CHUNK // 2, True))


def kernel(node_types, edge_dists, edge_index, params):
    f32 = jnp.float32
    i32 = jnp.int32
    src = edge_index[0]
    dst = edge_index[1]
    srcp = jnp.concatenate([src, jnp.zeros((EPAD - E,), i32)])
    dstp = jnp.concatenate([dst, jnp.full((EPAD - E,), V, i32)])
    src_c = srcp.reshape(EPAD // CHUNK, CHUNK)
    dst_c = dstp.reshape(EPAD // CHUNK, CHUNK)
    srcs1 = jnp.stack([src_c, src_c + VP])
    dsts1 = jnp.stack([dst_c, dst_c])
    nh = EPAD // CHUNK // 2
    srcs0 = jnp.stack([src_c[:nh], src_c[nh:] + VP])
    dsts0 = jnp.stack([dst_c[:nh], dst_c[nh:]])
    dist = edge_dists.astype(f32)
    ntp = jnp.concatenate([node_types, jnp.zeros((VP - V,), i32)])

    layers = params['layers']
    we1 = jnp.stack([jnp.pad(p['We1'], ((0, F - NCEN), (0, 0)))
                     for p in layers[1:]])
    be1 = jnp.stack([p['be1'].reshape(1, F) for p in layers[1:]])
    we2 = jnp.stack([p['We2'] for p in layers[1:]])
    be2 = jnp.stack([p['be2'].reshape(1, F) for p in layers[1:]])

    embed_k, msg_k, msg0_k = _sc_kernels()

    p0 = layers[0]
    tableP = pl.pallas_call(
        _prep_body,
        grid=(1,),
        in_specs=[_full((NUM_TYPES, F)), _full((F, F)), _full((1, F))],
        out_specs=_full((NUM_TYPES, 2 * F)),
        out_shape=jax.ShapeDtypeStruct((NUM_TYPES, 2 * F), f32),
    )(params['embed'], p0['Wn'], p0['bn'].reshape(1, F))

    hv = embed_k(tableP, ntp)

    he0 = pl.pallas_call(
        _edge1_body,
        grid=(E // EB,),
        in_specs=[
            pl.BlockSpec((EB, 1), lambda i: (i, 0)),
            _full((F, F)), _full((1, F)), _full((F, F)), _full((1, F)),
        ],
        out_specs=pl.BlockSpec((EB, F), lambda i: (i, 0)),
        out_shape=jax.ShapeDtypeStruct((EPAD, F), f32),
    )(dist, jnp.pad(p0['We1'], ((0, F - NCEN), (0, 0))),
      p0['be1'].reshape(1, F), p0['We2'], p0['be2'].reshape(1, F))

    he1, he2 = pl.pallas_call(
        _edge2_body,
        grid=(E // EB,),
        in_specs=[
            pl.BlockSpec((EB, 1), lambda i: (i, 0)),
            _full((2, F, F)), _full((2, 1, F)), _full((2, F, F)),
            _full((2, 1, F)),
        ],
        out_specs=[pl.BlockSpec((EB, F), lambda i: (i, 0))] * 2,
        out_shape=[jax.ShapeDtypeStruct((EPAD, F), f32)] * 2,
    )(dist, we1, be1, we2, be2)
    he = {0: he0, 1: he1, 2: he2}

    def tc_b0(h, p):
        return pl.pallas_call(
            _b0_body,
            grid=(VP // BB,),
            in_specs=[
                pl.BlockSpec((1, BB, 2 * F), lambda i: (0, i, 0)),
                _full((F, F)), _full((1, F)), _full((F, F)), _full((1, F)),
            ],
            out_specs=pl.BlockSpec((T, BB, F), lambda i: (0, i, 0)),
            out_shape=jax.ShapeDtypeStruct((T, VP, F), f32),
        )(h, p['Wc'], p['bc'].reshape(1, F), p['Wo'], p['bo'].reshape(1, F))

    def tc_b(h, p):
        return pl.pallas_call(
            _b_body,
            grid=(VP // BB,),
            in_specs=[
                pl.BlockSpec((2, BB, 2 * F), lambda i: (0, i, 0)),
                _full((F, F)), _full((1, F)), _full((F, F)), _full((1, F)),
            ],
            out_specs=pl.BlockSpec((T, BB, F), lambda i: (0, i, 0)),
            out_shape=jax.ShapeDtypeStruct((T, VP, F), f32),
        )(h, p['Wc'], p['bc'].reshape(1, F), p['Wo'], p['bo'].reshape(1, F))

    def tc_tm(o4, p):
        x2 = o4.reshape(T, N2)
        y = pl.pallas_call(
            _tm_body,
            grid=(N2 // NB,),
            in_specs=[
                pl.BlockSpec((T, NB), lambda i: (0, i)),
                _full((F, T)), _full((F, 1)), _full((T, F)), _full((T, 1)),
            ],
            out_specs=pl.BlockSpec((T, NB), lambda i: (0, i)),
            out_shape=jax.ShapeDtypeStruct((T, N2), f32),
        )(x2, p['Wt1'].T, p['bt1'].reshape(F, 1), p['Wt2'].T,
          p['bt2'].reshape(T, 1))
        return y.reshape(T, VP, F)

    def tc_proj(y4, p):
        return pl.pallas_call(
            _proj_body,
            grid=(VP // BB,),
            in_specs=[
                pl.BlockSpec((T, BB, F), lambda i: (0, i, 0)),
                _full((F, F)), _full((1, F)),
            ],
            out_specs=pl.BlockSpec((2, BB, 2 * F), lambda i: (0, i, 0)),
            out_shape=jax.ShapeDtypeStruct((2, VP, 2 * F), f32),
        )(y4, p['Wn'], p['bn'].reshape(1, F))

    he0c = he0.reshape(2, EPAD // 2, F)
    h0 = msg0_k(hv, he0c, srcs0, dsts0)
    y = tc_tm(tc_b0(h0, p0), p0)

    for l in (1, 2):
        p = layers[l]
        hv2 = tc_proj(y, p).reshape(2 * VP, 2 * F)
        hl = msg_k(hv2, he[l], srcs1, dsts1)
        if l == 1:
            y = tc_tm(tc_b(hl, p), p)
        else:
            o4 = tc_b(hl, p)

    x2 = o4.reshape(T, N2)
    p2 = layers[2]
    out = pl.pallas_call(
        _tmf_body,
        grid=(N2 // NB,),
        in_specs=[
            pl.BlockSpec((T, NB), lambda i: (0, i)),
            _full((F, T)), _full((F, 1)), _full((T, F)), _full((T, 1)),
            _full((32, T)), _full((32, 1)), _full((1, 32)), _full((1, 1)),
        ],
        out_specs=pl.BlockSpec((1, NB), lambda i: (0, i)),
        out_shape=jax.ShapeDtypeStruct((1, N2), f32),
    )(x2, p2['Wt1'].T, p2['bt1'].reshape(F, 1), p2['Wt2'].T,
      p2['bt2'].reshape(T, 1), params['Wr1'].T, params['br1'].reshape(32, 1),
      params['Wr2'].T, params['br2'].reshape(1, 1))

    return out.reshape(VP, F)[:V]
